# baseline (speedup 1.0000x reference)
import functools

import jax
import jax.numpy as jnp
from jax import lax
from jax.experimental import pallas as pl
from jax.experimental.pallas import tpu as pltpu
from jax.experimental.pallas import tpu_sc as plsc

EMB = 128
BATCH = 16384
NC = 2
NS = 16
NW = NC * NS
BPW = BATCH // NW
CHUNK = 128
NCHUNK = BPW // CHUNK

_mesh = plsc.VectorSubcoreMesh(core_axis_name="c", subcore_axis_name="s")


@functools.partial(
    pl.kernel,
    mesh=_mesh,
    out_type=jax.ShapeDtypeStruct((BATCH, 16), jnp.float32),
    scratch_types=[
        pltpu.VMEM((CHUNK,), jnp.int32),
        pltpu.VMEM((CHUNK,), jnp.int32),
        pltpu.VMEM((CHUNK, EMB), jnp.float32),
        pltpu.VMEM((CHUNK, EMB), jnp.float32),
        pltpu.VMEM((BPW, 16), jnp.float32),
        pltpu.SemaphoreType.DMA,
    ],
)
def _sc_dots(x1_hbm, x2_hbm, emb_hbm, ctx_hbm, out_hbm,
             idx1_v, idx2_v, rows1, rows2, pacc_v, sem):
    wid = lax.axis_index("s") * NC + lax.axis_index("c")
    base = wid * BPW
    for ci in range(NCHUNK):
        off = base + ci * CHUNK
        pltpu.sync_copy(x1_hbm.at[pl.ds(off, CHUNK)], idx1_v)
        pltpu.sync_copy(x2_hbm.at[pl.ds(off, CHUNK)], idx2_v)
        cp1 = pltpu.async_copy(emb_hbm.at[idx1_v], rows1, sem)
        cp2 = pltpu.async_copy(ctx_hbm.at[idx2_v], rows2, sem)
        cp1.wait()
        cp2.wait()

        def row_body(r, _, ci=ci):
            acc = rows1[r, pl.ds(0, 16)] * rows2[r, pl.ds(0, 16)]
            for j in range(1, EMB // 16):
                acc = acc + rows1[r, pl.ds(j * 16, 16)] * rows2[r, pl.ds(j * 16, 16)]
            pacc_v[ci * CHUNK + r, :] = acc
            return 0

        lax.fori_loop(0, CHUNK, row_body, 0)
    pltpu.sync_copy(pacc_v, out_hbm.at[pl.ds(base, BPW)])


def _loss_body(p_ref, w_ref, o_ref):
    s = jnp.sum(p_ref[...], axis=1, keepdims=True)
    x = w_ref[...] * s
    ls = jnp.minimum(x, 0.0) - jnp.log1p(jnp.exp(-jnp.abs(x)))
    o_ref[0, 0] = -jnp.mean(ls)


def _tc_loss(pacc, w2d):
    return pl.pallas_call(
        _loss_body,
        out_shape=jax.ShapeDtypeStruct((1, 1), jnp.float32),
        out_specs=pl.BlockSpec(memory_space=pltpu.SMEM),
    )(pacc, w2d)


def kernel(x1, x2, w, emb_w, ctx_w):
    x1 = x1.astype(jnp.int32)
    x2 = x2.astype(jnp.int32)
    pacc = _sc_dots(x1, x2, emb_w, ctx_w)
    loss = _tc_loss(pacc, w.astype(jnp.float32).reshape(BATCH, 1))
    return loss.reshape(())


# speedup vs baseline: 1.0064x; 1.0064x over previous
import functools

import jax
import jax.numpy as jnp
from jax import lax
from jax.experimental import pallas as pl
from jax.experimental.pallas import tpu as pltpu
from jax.experimental.pallas import tpu_sc as plsc

EMB = 128
BATCH = 16384
NC = 2
NS = 16
NW = NC * NS
BPW = BATCH // NW
CHUNK = 64
NCHUNK = BPW // CHUNK

_mesh = plsc.VectorSubcoreMesh(core_axis_name="c", subcore_axis_name="s")


@functools.partial(
    pl.kernel,
    mesh=_mesh,
    out_type=jax.ShapeDtypeStruct((BATCH, 16), jnp.float32),
    scratch_types=[
        pltpu.VMEM((CHUNK,), jnp.int32),
        pltpu.VMEM((CHUNK,), jnp.int32),
        pltpu.VMEM((CHUNK,), jnp.int32),
        pltpu.VMEM((CHUNK,), jnp.int32),
        pltpu.VMEM((CHUNK, EMB), jnp.float32),
        pltpu.VMEM((CHUNK, EMB), jnp.float32),
        pltpu.VMEM((CHUNK, EMB), jnp.float32),
        pltpu.VMEM((CHUNK, EMB), jnp.float32),
        pltpu.VMEM((BPW, 16), jnp.float32),
        pltpu.SemaphoreType.DMA,
        pltpu.SemaphoreType.DMA,
    ],
)
def _sc_dots(x1_hbm, x2_hbm, emb_hbm, ctx_hbm, out_hbm,
             idx1_a, idx2_a, idx1_b, idx2_b,
             rows1_a, rows2_a, rows1_b, rows2_b,
             pacc_v, sem_a, sem_b):
    wid = lax.axis_index("s") * NC + lax.axis_index("c")
    base = wid * BPW
    bufs = [(idx1_a, idx2_a, rows1_a, rows2_a, sem_a),
            (idx1_b, idx2_b, rows1_b, rows2_b, sem_b)]

    def _start(ci, buf):
        idx1, idx2, r1, r2, sem = buf
        off = base + ci * CHUNK
        pltpu.sync_copy(x1_hbm.at[pl.ds(off, CHUNK)], idx1)
        pltpu.sync_copy(x2_hbm.at[pl.ds(off, CHUNK)], idx2)
        c1 = pltpu.async_copy(emb_hbm.at[idx1], r1, sem)
        c2 = pltpu.async_copy(ctx_hbm.at[idx2], r2, sem)
        return c1, c2

    pend = [None, None]
    pend[0] = _start(0, bufs[0])
    for ci in range(NCHUNK):
        p = ci % 2
        if ci + 1 < NCHUNK:
            pend[1 - p] = _start(ci + 1, bufs[1 - p])
        c1, c2 = pend[p]
        c1.wait()
        c2.wait()
        r1, r2 = bufs[p][2], bufs[p][3]

        def row4_body(i, _, ci=ci, r1=r1, r2=r2):
            for k in range(4):
                r = i * 4 + k
                acc = r1[r, pl.ds(0, 16)] * r2[r, pl.ds(0, 16)]
                for j in range(1, EMB // 16):
                    acc = acc + r1[r, pl.ds(j * 16, 16)] * r2[r, pl.ds(j * 16, 16)]
                pacc_v[ci * CHUNK + r, :] = acc
            return 0

        lax.fori_loop(0, CHUNK // 4, row4_body, 0)
    pltpu.sync_copy(pacc_v, out_hbm.at[pl.ds(base, BPW)])


def _loss_body(p_ref, w_ref, o_ref):
    s = jnp.sum(p_ref[...], axis=1, keepdims=True)
    x = w_ref[...] * s
    ls = jnp.minimum(x, 0.0) - jnp.log1p(jnp.exp(-jnp.abs(x)))
    o_ref[0, 0] = -jnp.mean(ls)


def _tc_loss(pacc, w2d):
    return pl.pallas_call(
        _loss_body,
        out_shape=jax.ShapeDtypeStruct((1, 1), jnp.float32),
        out_specs=pl.BlockSpec(memory_space=pltpu.SMEM),
    )(pacc, w2d)


def kernel(x1, x2, w, emb_w, ctx_w):
    x1 = x1.astype(jnp.int32)
    x2 = x2.astype(jnp.int32)
    pacc = _sc_dots(x1, x2, emb_w, ctx_w)
    loss = _tc_loss(pacc, w.astype(jnp.float32).reshape(BATCH, 1))
    return loss.reshape(())


# speedup vs baseline: 1.0129x; 1.0064x over previous
import functools

import jax
import jax.numpy as jnp
from jax import lax
from jax.experimental import pallas as pl
from jax.experimental.pallas import tpu as pltpu
from jax.experimental.pallas import tpu_sc as plsc

EMB = 128
BATCH = 16384
NC = 2
NS = 16
NW = NC * NS
BPW = BATCH // NW
CHUNK = 64
NCHUNK = BPW // CHUNK

_mesh = plsc.VectorSubcoreMesh(core_axis_name="c", subcore_axis_name="s")


@functools.partial(
    pl.kernel,
    mesh=_mesh,
    out_type=jax.ShapeDtypeStruct((BATCH, 16), jnp.float32),
    scratch_types=[
        pltpu.VMEM((CHUNK,), jnp.int32),
        pltpu.VMEM((CHUNK,), jnp.int32),
        pltpu.VMEM((CHUNK,), jnp.int32),
        pltpu.VMEM((CHUNK,), jnp.int32),
        pltpu.VMEM((CHUNK, EMB), jnp.float32),
        pltpu.VMEM((CHUNK, EMB), jnp.float32),
        pltpu.VMEM((CHUNK, EMB), jnp.float32),
        pltpu.VMEM((CHUNK, EMB), jnp.float32),
        pltpu.VMEM((BPW, 16), jnp.float32),
        pltpu.SemaphoreType.DMA,
        pltpu.SemaphoreType.DMA,
    ],
)
def _sc_dots(x1_hbm, x2_hbm, emb_hbm, ctx_hbm, out_hbm,
             idx1_a, idx2_a, idx1_b, idx2_b,
             rows1_a, rows2_a, rows1_b, rows2_b,
             pacc_v, sem_a, sem_b):
    wid = lax.axis_index("s") * NC + lax.axis_index("c")
    base = wid * BPW
    bufs = [(idx1_a, idx2_a, rows1_a, rows2_a, sem_a),
            (idx1_b, idx2_b, rows1_b, rows2_b, sem_b)]

    def _start(ci, buf):
        idx1, idx2, r1, r2, sem = buf
        off = base + ci * CHUNK
        pltpu.sync_copy(x1_hbm.at[pl.ds(off, CHUNK)], idx1)
        pltpu.sync_copy(x2_hbm.at[pl.ds(off, CHUNK)], idx2)
        c1 = pltpu.async_copy(emb_hbm.at[idx1], r1, sem)
        c2 = pltpu.async_copy(ctx_hbm.at[idx2], r2, sem)
        return c1, c2

    pend = [None, None]
    pend[0] = _start(0, bufs[0])
    for ci in range(NCHUNK):
        p = ci % 2
        if ci + 1 < NCHUNK:
            pend[1 - p] = _start(ci + 1, bufs[1 - p])
        c1, c2 = pend[p]
        c1.wait()
        c2.wait()
        r1, r2 = bufs[p][2], bufs[p][3]

        def row4_body(i, _, ci=ci, r1=r1, r2=r2):
            for k in range(4):
                r = i * 4 + k
                acc = r1[r, pl.ds(0, 16)] * r2[r, pl.ds(0, 16)]
                for j in range(1, EMB // 16):
                    acc = acc + r1[r, pl.ds(j * 16, 16)] * r2[r, pl.ds(j * 16, 16)]
                pacc_v[ci * CHUNK + r, :] = acc
            return 0

        lax.fori_loop(0, CHUNK // 4, row4_body, 0)
    pltpu.sync_copy(pacc_v, out_hbm.at[pl.ds(base, BPW)])


def _loss_body(p_ref, w_ref, o_ref):
    sel = (lax.broadcasted_iota(jnp.int32, (2048, 128), 0) // 16
           == lax.broadcasted_iota(jnp.int32, (2048, 128), 1)).astype(jnp.float32)
    d = lax.dot_general(p_ref[...], sel, (((1,), (0,)), ((), ())),
                        precision=lax.Precision.HIGHEST,
                        preferred_element_type=jnp.float32)
    x = w_ref[...] * d
    ls = jnp.minimum(x, 0.0) - jnp.log1p(jnp.exp(-jnp.abs(x)))
    o_ref[0, 0] = -jnp.mean(ls)


def _tc_loss(pacc, w2d):
    return pl.pallas_call(
        _loss_body,
        out_shape=jax.ShapeDtypeStruct((1, 1), jnp.float32),
        out_specs=pl.BlockSpec(memory_space=pltpu.SMEM),
    )(pacc, w2d)


def kernel(x1, x2, w, emb_w, ctx_w):
    x1 = x1.astype(jnp.int32)
    x2 = x2.astype(jnp.int32)
    pacc = _sc_dots(x1, x2, emb_w, ctx_w)
    loss = _tc_loss(pacc.reshape(128, 2048),
                    w.astype(jnp.float32).reshape(128, 128))
    return loss.reshape(())
